# baseline (speedup 1.0000x reference)
import functools

import jax
import jax.numpy as jnp
from jax import lax
from jax.experimental import pallas as pl
from jax.experimental.pallas import tpu as pltpu
from jax.experimental.pallas import tpu_sc as plsc

INPUT_DIM = 256
NUM_RISK_BINS = 4
NUM_ENG_BINS = 4
NUM_TRANSIENT = 7
DROPOUT_STATE_ID = 24
NUM_BASE_STATES = NUM_RISK_BINS * NUM_ENG_BINS

BLK = 8192


def _scores_kernel(h_ref, wt_ref, bias_ref, w2t_ref, b2t_ref, out_ref):
    h = h_ref[...]
    st = lax.dot_general(wt_ref[...], h, (((1,), (1,)), ((), ())),
                         preferred_element_type=jnp.float32)
    st = st + bias_ref[...]
    out_ref[0:1, :] = jax.nn.sigmoid(st[64:65, :])
    out_ref[1:2, :] = jax.nn.sigmoid(st[65:66, :])
    hidden = jnp.maximum(st[0:64, :], 0.0)
    logits = jnp.dot(w2t_ref[...], hidden, preferred_element_type=jnp.float32)
    logits = logits + b2t_ref[...]
    out_ref[2:10, :] = logits


def _tc_scores(h, risk_W, risk_b, eng_W, eng_b, t_W1, t_b1, t_W2, t_b2):
    B = h.shape[0]
    nblk = B // BLK
    wt = jnp.concatenate(
        [t_W1.T, risk_W.T, eng_W.T, jnp.zeros((62, INPUT_DIM), jnp.float32)], axis=0)
    bias = jnp.concatenate(
        [t_b1, risk_b, eng_b, jnp.zeros((62,), jnp.float32)]).reshape(128, 1)
    w2t = jnp.concatenate([t_W2.T, jnp.zeros((1, 64), jnp.float32)], axis=0)
    b2t = jnp.concatenate([t_b2, jnp.full((1,), -1e30, jnp.float32)]).reshape(8, 1)

    return pl.pallas_call(
        _scores_kernel,
        grid=(nblk,),
        in_specs=[
            pl.BlockSpec((BLK, INPUT_DIM), lambda i: (i, 0)),
            pl.BlockSpec((128, INPUT_DIM), lambda i: (0, 0)),
            pl.BlockSpec((128, 1), lambda i: (0, 0)),
            pl.BlockSpec((8, 64), lambda i: (0, 0)),
            pl.BlockSpec((8, 1), lambda i: (0, 0)),
        ],
        out_specs=pl.BlockSpec((16, BLK), lambda i: (0, i)),
        out_shape=jax.ShapeDtypeStruct((16, B), jnp.float32),
    )(h, wt, bias, w2t, b2t)


def _sc_discretize(scores, dl):
    B = scores.shape[1]
    info = plsc.get_sparse_core_info()
    nc, ns, lanes = info.num_cores, info.num_subcores, info.num_lanes
    nw = nc * ns
    bpw = B // nw
    nchunk = bpw // lanes
    mesh = plsc.VectorSubcoreMesh(core_axis_name="c", subcore_axis_name="s")

    @functools.partial(
        pl.kernel, mesh=mesh,
        out_type=jax.ShapeDtypeStruct((B,), jnp.int32),
        scratch_types=[
            pltpu.VMEM((16, bpw), jnp.float32),
            pltpu.VMEM((bpw,), jnp.int32),
            pltpu.VMEM((bpw,), jnp.int32),
        ],
    )
    def sc_k(scores_hbm, dl_hbm, out_hbm, sc_v, dl_v, out_v):
        wid = lax.axis_index("s") * nc + lax.axis_index("c")
        base = wid * bpw
        pltpu.sync_copy(scores_hbm.at[:, pl.ds(base, bpw)], sc_v)
        pltpu.sync_copy(dl_hbm.at[pl.ds(base, bpw)], dl_v)
        for i in range(nchunk):
            sl = pl.ds(i * lanes, lanes)
            r = sc_v[0, sl]
            e = sc_v[1, sl]
            m = sc_v[2, sl]
            idx = jnp.zeros((lanes,), jnp.int32)
            for j in range(1, NUM_TRANSIENT):
                lj = sc_v[2 + j, sl]
                gt = lj > m
                idx = jnp.where(gt, jnp.full((lanes,), j, jnp.int32), idx)
                m = jnp.where(gt, lj, m)
            rb = jnp.clip((r * NUM_RISK_BINS).astype(jnp.int32), 0, NUM_RISK_BINS - 1)
            eb = jnp.clip((e * NUM_ENG_BINS).astype(jnp.int32), 0, NUM_ENG_BINS - 1)
            fin = jnp.where(r > 0.75, NUM_BASE_STATES + idx,
                            rb * NUM_ENG_BINS + eb)
            fin = jnp.where(dl_v[sl] == 1, jnp.int32(DROPOUT_STATE_ID), fin)
            out_v[sl] = fin
        pltpu.sync_copy(out_v, out_hbm.at[pl.ds(base, bpw)])

    return sc_k(scores, dl)


def kernel(h, dropout_labels, risk_W, risk_b, eng_W, eng_b, t_W1, t_b1, t_W2, t_b2):
    B = h.shape[0]
    half = B // 2
    dl = dropout_labels.astype(jnp.int32)
    scores0 = _tc_scores(h[:half], risk_W, risk_b, eng_W, eng_b,
                         t_W1, t_b1, t_W2, t_b2)
    scores1 = _tc_scores(h[half:], risk_W, risk_b, eng_W, eng_b,
                         t_W1, t_b1, t_W2, t_b2)
    out0 = _sc_discretize(scores0, dl[:half])
    out1 = _sc_discretize(scores1, dl[half:])
    return jnp.concatenate([out0, out1])


# speedup vs baseline: 3.2299x; 3.2299x over previous
import jax
import jax.numpy as jnp
from jax import lax
from jax.experimental import pallas as pl
from jax.experimental.pallas import tpu as pltpu

INPUT_DIM = 256
NUM_RISK_BINS = 4
NUM_ENG_BINS = 4
NUM_TRANSIENT = 7
DROPOUT_STATE_ID = 24
NUM_BASE_STATES = NUM_RISK_BINS * NUM_ENG_BINS

BLK = 8192


def _disc_kernel(h_ref, dl_ref, wt_ref, bias_ref, w2t_ref, b2t_ref, out_ref):
    h = h_ref[...]
    st = lax.dot_general(wt_ref[...], h, (((1,), (1,)), ((), ())),
                         preferred_element_type=jnp.float32)
    st = st + bias_ref[...]
    risk = jax.nn.sigmoid(st[64:65, :])
    eng = jax.nn.sigmoid(st[65:66, :])
    hidden = jnp.maximum(st[0:64, :], 0.0)
    logits = jnp.dot(w2t_ref[...], hidden, preferred_element_type=jnp.float32)
    logits = logits + b2t_ref[...]
    tstate = jnp.argmax(logits, axis=0).astype(jnp.int32)
    rbin = jnp.clip((risk * NUM_RISK_BINS).astype(jnp.int32), 0, NUM_RISK_BINS - 1)
    ebin = jnp.clip((eng * NUM_ENG_BINS).astype(jnp.int32), 0, NUM_ENG_BINS - 1)
    base = (rbin * NUM_ENG_BINS + ebin)[0, :]
    final = jnp.where(risk[0, :] > 0.75, NUM_BASE_STATES + tstate, base)
    final = jnp.where(dl_ref[0, 0, :] == 1, jnp.int32(DROPOUT_STATE_ID), final)
    out_ref[0, 0, :] = final


def kernel(h, dropout_labels, risk_W, risk_b, eng_W, eng_b, t_W1, t_b1, t_W2, t_b2):
    B = h.shape[0]
    nblk = B // BLK

    wt = jnp.concatenate(
        [t_W1.T, risk_W.T, eng_W.T, jnp.zeros((62, INPUT_DIM), jnp.float32)], axis=0)
    bias = jnp.concatenate(
        [t_b1, risk_b, eng_b, jnp.zeros((62,), jnp.float32)]).reshape(128, 1)
    w2t = jnp.concatenate([t_W2.T, jnp.zeros((1, 64), jnp.float32)], axis=0)
    b2t = jnp.concatenate([t_b2, jnp.full((1,), -1e30, jnp.float32)]).reshape(8, 1)

    dl = dropout_labels.astype(jnp.int32).reshape(nblk, 1, BLK)

    out = pl.pallas_call(
        _disc_kernel,
        grid=(nblk,),
        in_specs=[
            pl.BlockSpec((BLK, INPUT_DIM), lambda i: (i, 0)),
            pl.BlockSpec((1, 1, BLK), lambda i: (i, 0, 0)),
            pl.BlockSpec((128, INPUT_DIM), lambda i: (0, 0)),
            pl.BlockSpec((128, 1), lambda i: (0, 0)),
            pl.BlockSpec((8, 64), lambda i: (0, 0)),
            pl.BlockSpec((8, 1), lambda i: (0, 0)),
        ],
        out_specs=pl.BlockSpec((1, 1, BLK), lambda i: (i, 0, 0)),
        out_shape=jax.ShapeDtypeStruct((nblk, 1, BLK), jnp.int32),
    )(h, dl, wt, bias, w2t, b2t)
    return out.reshape(B)
